# baseline (speedup 1.0000x reference)
import jax
import jax.numpy as jnp
from jax import lax
from jax.experimental import pallas as pl
from jax.experimental.pallas import tpu as pltpu
from jax.experimental.pallas import tpu_sc as plsc

N = 320000
G = 128
M = 10000

NC = 2
NS = 16
NW = NC * NS
SPT = 320
CHUNK = 256
L = 16
GV = G // L
BIG = 3.0e38


def _sc_body(cost_hbm, ids_hbm, starts_hbm, params_hbm, map_out, mask_out,
             acc, cntf, maskbuf, ids_a, ids_b, cost_a, cost_b, starts_v,
             params_v, sem_a, sem_b):
    wid = lax.axis_index("s") * NC + lax.axis_index("c")
    s0 = wid * SPT

    pltpu.sync_copy(starts_hbm, starts_v)
    pltpu.sync_copy(params_hbm, params_v)
    default_v = params_v[0:L]
    resid_v = params_v[L:2 * L]

    p0 = starts_v[pl.ds(wid, L)][0]
    p1 = starts_v[pl.ds(wid + 1, L)][0]

    big_v = jnp.full((L,), BIG, jnp.float32)
    zero_v = jnp.zeros((L,), jnp.float32)
    lane_v = lax.broadcasted_iota(jnp.int32, (L,), 0)
    one0_v = (1 - jnp.minimum(lane_v, 1)).astype(jnp.float32)

    def init_body(s, _):
        for r in range(GV):
            acc[s, pl.ds(r * L, L)] = big_v
        return 0
    lax.fori_loop(0, SPT, init_body, 0, unroll=4)

    def cnt_init(i, _):
        cntf[pl.ds(i * L, L)] = zero_v
        return 0
    lax.fori_loop(0, (SPT + L) // L, cnt_init, 0, unroll=4)

    j0 = p0 // CHUNK
    j1 = (p1 + CHUNK - 1) // CHUNK

    def run_chunk(j, idsb, costb, carry):
        off = j * CHUNK
        idsb[pl.ds(0, L)] = lax.broadcast_in_dim(carry[1], (L,), ())
        lo = jnp.maximum(p0 - off, 0)
        hi = jnp.minimum(p1, off + CHUNK) - off

        def point(li, cnt, crow):
            pv = idsb[pl.ds(L - 1 + li, L)]
            ls = pv[1] - s0
            kif = lax.convert_element_type(
                jnp.minimum(jnp.abs(pv[1] - pv[0]), 1), jnp.float32)
            kbv = lax.broadcast_in_dim(kif * BIG, (L,), ())
            new = [jnp.minimum(crow[r] + kbv,
                               costb[li, pl.ds(r * L, L)])
                   for r in range(GV)]
            for r in range(GV):
                acc[ls, pl.ds(r * L, L)] = new[r]
            ncnt = cnt * (1.0 - kif) + 1.0
            cntf[pl.ds(ls, L)] = lax.broadcast_in_dim(ncnt, (L,), ()) * one0_v
            return ncnt, new

        def pt_body(li, pc):
            ncnt, new = point(li, pc[0], pc[1:])
            return (ncnt, *new)

        def blk_body(g, pc):
            base = g * L
            idv = idsb[pl.ds(L + base, L)]
            pidv = idsb[pl.ds(L - 1 + base, L)]
            kfg = lax.convert_element_type(
                jnp.minimum(jnp.abs(idv - pidv), 1), jnp.float32)
            kbg = kfg * BIG
            cnt = pc[0]
            crow = pc[1:]
            for i in range(0, L, 2):
                ls1 = idv[i] - s0
                ls2 = idv[i + 1] - s0
                kb1 = lax.broadcast_in_dim(kbg[i], (L,), ())
                kb2 = lax.broadcast_in_dim(kbg[i + 1], (L,), ())
                kbs = lax.broadcast_in_dim(kbg[i] + kbg[i + 1], (L,), ())
                c1 = [costb[base + i, pl.ds(r * L, L)] for r in range(GV)]
                c2 = [costb[base + i + 1, pl.ds(r * L, L)]
                      for r in range(GV)]
                t = [jnp.minimum(c1[r] + kb2, c2[r]) for r in range(GV)]
                st1 = [jnp.minimum(crow[r] + kb1, c1[r]) for r in range(GV)]
                crow = [jnp.minimum(crow[r] + kbs, t[r]) for r in range(GV)]
                for r in range(GV):
                    acc[ls1, pl.ds(r * L, L)] = st1[r]
                for r in range(GV):
                    acc[ls2, pl.ds(r * L, L)] = crow[r]
                cnt = cnt * (1.0 - kfg[i]) + 1.0
                cntf[pl.ds(ls1, L)] = (lax.broadcast_in_dim(cnt, (L,), ())
                                       * one0_v)
                cnt = cnt * (1.0 - kfg[i + 1]) + 1.0
                cntf[pl.ds(ls2, L)] = (lax.broadcast_in_dim(cnt, (L,), ())
                                       * one0_v)
            return (cnt, *crow)

        a = jnp.minimum(hi, ((lo + L - 1) // L) * L)
        b = jnp.maximum(a, (hi // L) * L)
        pc = (carry[0],) + carry[2:]
        pc = lax.fori_loop(lo, a, pt_body, pc)
        pc = lax.fori_loop(a // L, b // L, blk_body, pc)
        pc = lax.fori_loop(b, hi, pt_body, pc)
        nprev = idsb[pl.ds(CHUNK, L)][L - 1]
        return (pc[0], nprev) + pc[1:]

    def clamp_off(j):
        return jnp.clip(j, 0, jnp.maximum(j1 - 1, 0)) * CHUNK

    def issue(j, idsb, costb, sem):
        off = clamp_off(j)
        pltpu.async_copy(ids_hbm.at[pl.ds(off, CHUNK)],
                         idsb.at[pl.ds(L, CHUNK)], sem)
        pltpu.async_copy(cost_hbm.at[pl.ds(off, CHUNK)], costb, sem)

    def drain(j, idsb, costb, sem):
        off = clamp_off(j)
        pltpu.make_async_copy(ids_hbm.at[pl.ds(off, CHUNK)],
                              idsb.at[pl.ds(L, CHUNK)], sem).wait()
        pltpu.make_async_copy(cost_hbm.at[pl.ds(off, CHUNK)], costb,
                              sem).wait()

    issue(j0, ids_a, cost_a, sem_a)
    issue(j0 + 1, ids_b, cost_b, sem_b)

    def pair_body(t, pc):
        j = j0 + 2 * t
        drain(j, ids_a, cost_a, sem_a)
        pc = run_chunk(j, ids_a, cost_a, pc)
        issue(j + 2, ids_a, cost_a, sem_a)
        drain(j + 1, ids_b, cost_b, sem_b)
        pc = run_chunk(j + 1, ids_b, cost_b, pc)
        issue(j + 3, ids_b, cost_b, sem_b)
        return pc

    carry0 = (jnp.float32(0.0), jnp.int32(-1)) + (big_v,) * GV
    nt = (j1 - j0 + 1) // 2
    pc = lax.fori_loop(0, nt, pair_body, carry0)
    jend = j0 + 2 * nt
    drain(jend, ids_a, cost_a, sem_a)
    drain(jend + 1, ids_b, cost_b, sem_b)

    def mask_body(i, _):
        cv = cntf[pl.ds(i * L, L)]
        maskbuf[pl.ds(i * L, L)] = cv - 1.0 + resid_v
        return 0
    lax.fori_loop(0, SPT // L, mask_body, 0, unroll=4)

    def sel_body(s, _):
        mv = maskbuf[pl.ds(s, L)]
        kf = jnp.clip(lax.broadcast_in_dim(mv[0], (L,), ()) + 1.0, 0.0, 1.0)
        for r in range(GV):
            a = acc[s, pl.ds(r * L, L)]
            acc[s, pl.ds(r * L, L)] = kf * a + (1.0 - kf) * default_v
        return 0
    lax.fori_loop(0, SPT, sel_body, 0, unroll=2)

    pltpu.sync_copy(acc, map_out.at[pl.ds(s0, SPT)])
    pltpu.sync_copy(maskbuf.at[pl.ds(0, SPT)], mask_out.at[pl.ds(s0, SPT)])


def kernel(cost, segment_ids, num_cells, default_cost):
    bounds = jnp.arange(NW + 1, dtype=jnp.int32) * SPT
    starts = jnp.searchsorted(segment_ids, bounds, side="left").astype(jnp.int32)
    starts = jnp.pad(starts, (0, 64 - (NW + 1)))
    resid = (jnp.asarray(num_cells) - M).astype(jnp.float32)
    params = jnp.concatenate([
        jnp.full((L,), default_cost, jnp.float32),
        jnp.full((L,), resid, jnp.float32),
    ])

    mesh = plsc.VectorSubcoreMesh(
        core_axis_name="c", subcore_axis_name="s",
        num_cores=NC, num_subcores=NS)
    map_pad, mask_pad = pl.kernel(
        _sc_body,
        out_type=[
            jax.ShapeDtypeStruct((NW * SPT, G), jnp.float32),
            jax.ShapeDtypeStruct((NW * SPT,), jnp.float32),
        ],
        mesh=mesh,
        scratch_types=[
            pltpu.VMEM((SPT, G), jnp.float32),
            pltpu.VMEM((SPT + L,), jnp.float32),
            pltpu.VMEM((SPT + L,), jnp.float32),
            pltpu.VMEM((CHUNK + 2 * L,), jnp.int32),
            pltpu.VMEM((CHUNK + 2 * L,), jnp.int32),
            pltpu.VMEM((CHUNK, G), jnp.float32),
            pltpu.VMEM((CHUNK, G), jnp.float32),
            pltpu.VMEM((64,), jnp.int32),
            pltpu.VMEM((2 * L,), jnp.float32),
            pltpu.SemaphoreType.DMA,
            pltpu.SemaphoreType.DMA,
        ],
    )(cost, segment_ids, starts, params)
    return map_pad[:M], mask_pad[:M]


# speedup vs baseline: 1.0049x; 1.0049x over previous
import jax
import jax.numpy as jnp
from jax import lax
from jax.experimental import pallas as pl
from jax.experimental.pallas import tpu as pltpu
from jax.experimental.pallas import tpu_sc as plsc

N = 320000
G = 128
M = 10000

NC = 2
NS = 16
NW = NC * NS
SPT = 320
CHUNK = 320
L = 16
GV = G // L
BIG = 3.0e38


def _sc_body(cost_hbm, ids_hbm, starts_hbm, params_hbm, map_out, mask_out,
             acc, cntf, maskbuf, ids_a, ids_b, cost_a, cost_b, starts_v,
             params_v, sem_a, sem_b):
    wid = lax.axis_index("s") * NC + lax.axis_index("c")
    s0 = wid * SPT

    pltpu.sync_copy(starts_hbm, starts_v)
    pltpu.sync_copy(params_hbm, params_v)
    default_v = params_v[0:L]
    resid_v = params_v[L:2 * L]

    p0 = starts_v[pl.ds(wid, L)][0]
    p1 = starts_v[pl.ds(wid + 1, L)][0]

    big_v = jnp.full((L,), BIG, jnp.float32)
    zero_v = jnp.zeros((L,), jnp.float32)
    lane_v = lax.broadcasted_iota(jnp.int32, (L,), 0)
    one0_v = (1 - jnp.minimum(lane_v, 1)).astype(jnp.float32)

    def init_body(s, _):
        for r in range(GV):
            acc[s, pl.ds(r * L, L)] = big_v
        return 0
    lax.fori_loop(0, SPT, init_body, 0, unroll=4)

    def cnt_init(i, _):
        cntf[pl.ds(i * L, L)] = zero_v
        return 0
    lax.fori_loop(0, (SPT + L) // L, cnt_init, 0, unroll=4)

    j0 = p0 // CHUNK
    j1 = (p1 + CHUNK - 1) // CHUNK

    def run_chunk(j, idsb, costb, carry):
        off = j * CHUNK
        idsb[pl.ds(0, L)] = lax.broadcast_in_dim(carry[1], (L,), ())
        lo = jnp.maximum(p0 - off, 0)
        hi = jnp.minimum(p1, off + CHUNK) - off

        def point(li, cnt, crow):
            pv = idsb[pl.ds(L - 1 + li, L)]
            ls = pv[1] - s0
            kif = lax.convert_element_type(
                jnp.minimum(jnp.abs(pv[1] - pv[0]), 1), jnp.float32)
            kbv = lax.broadcast_in_dim(kif * BIG, (L,), ())
            new = [jnp.minimum(crow[r] + kbv,
                               costb[li, pl.ds(r * L, L)])
                   for r in range(GV)]
            for r in range(GV):
                acc[ls, pl.ds(r * L, L)] = new[r]
            ncnt = cnt * (1.0 - kif) + 1.0
            cntf[pl.ds(ls, L)] = lax.broadcast_in_dim(ncnt, (L,), ()) * one0_v
            return ncnt, new

        def pt_body(li, pc):
            ncnt, new = point(li, pc[0], pc[1:])
            return (ncnt, *new)

        def blk_body(g, pc):
            base = g * L
            idv = idsb[pl.ds(L + base, L)]
            pidv = idsb[pl.ds(L - 1 + base, L)]
            kfg = lax.convert_element_type(
                jnp.minimum(jnp.abs(idv - pidv), 1), jnp.float32)
            kbg = kfg * BIG
            cnt = pc[0]
            crow = pc[1:]
            for i in range(L):
                ls = idv[i] - s0
                kbv = lax.broadcast_in_dim(kbg[i], (L,), ())
                crow = [jnp.minimum(crow[r] + kbv,
                                    costb[base + i, pl.ds(r * L, L)])
                        for r in range(GV)]
                for r in range(GV):
                    acc[ls, pl.ds(r * L, L)] = crow[r]
                cnt = cnt * (1.0 - kfg[i]) + 1.0
                cntf[pl.ds(ls, L)] = (lax.broadcast_in_dim(cnt, (L,), ())
                                      * one0_v)
            return (cnt, *crow)

        a = jnp.minimum(hi, ((lo + L - 1) // L) * L)
        b = jnp.maximum(a, (hi // L) * L)
        pc = (carry[0],) + carry[2:]
        pc = lax.fori_loop(lo, a, pt_body, pc)
        pc = lax.fori_loop(a // L, b // L, blk_body, pc)
        pc = lax.fori_loop(b, hi, pt_body, pc)
        nprev = idsb[pl.ds(CHUNK, L)][L - 1]
        return (pc[0], nprev) + pc[1:]

    def clamp_off(j):
        return jnp.clip(j, 0, jnp.maximum(j1 - 1, 0)) * CHUNK

    def issue(j, idsb, costb, sem):
        off = clamp_off(j)
        pltpu.async_copy(ids_hbm.at[pl.ds(off, CHUNK)],
                         idsb.at[pl.ds(L, CHUNK)], sem)
        pltpu.async_copy(cost_hbm.at[pl.ds(off, CHUNK)], costb, sem)

    def drain(j, idsb, costb, sem):
        off = clamp_off(j)
        pltpu.make_async_copy(ids_hbm.at[pl.ds(off, CHUNK)],
                              idsb.at[pl.ds(L, CHUNK)], sem).wait()
        pltpu.make_async_copy(cost_hbm.at[pl.ds(off, CHUNK)], costb,
                              sem).wait()

    issue(j0, ids_a, cost_a, sem_a)
    issue(j0 + 1, ids_b, cost_b, sem_b)

    def pair_body(t, pc):
        j = j0 + 2 * t
        drain(j, ids_a, cost_a, sem_a)
        pc = run_chunk(j, ids_a, cost_a, pc)
        issue(j + 2, ids_a, cost_a, sem_a)
        drain(j + 1, ids_b, cost_b, sem_b)
        pc = run_chunk(j + 1, ids_b, cost_b, pc)
        issue(j + 3, ids_b, cost_b, sem_b)
        return pc

    carry0 = (jnp.float32(0.0), jnp.int32(-1)) + (big_v,) * GV
    nt = (j1 - j0 + 1) // 2
    pc = lax.fori_loop(0, nt, pair_body, carry0)
    jend = j0 + 2 * nt
    drain(jend, ids_a, cost_a, sem_a)
    drain(jend + 1, ids_b, cost_b, sem_b)

    def mask_body(i, _):
        cv = cntf[pl.ds(i * L, L)]
        maskbuf[pl.ds(i * L, L)] = cv - 1.0 + resid_v
        return 0
    lax.fori_loop(0, SPT // L, mask_body, 0, unroll=4)

    def sel_body(s, _):
        mv = maskbuf[pl.ds(s, L)]
        kf = jnp.clip(lax.broadcast_in_dim(mv[0], (L,), ()) + 1.0, 0.0, 1.0)
        for r in range(GV):
            a = acc[s, pl.ds(r * L, L)]
            acc[s, pl.ds(r * L, L)] = kf * a + (1.0 - kf) * default_v
        return 0
    lax.fori_loop(0, SPT, sel_body, 0, unroll=2)

    pltpu.sync_copy(acc, map_out.at[pl.ds(s0, SPT)])
    pltpu.sync_copy(maskbuf.at[pl.ds(0, SPT)], mask_out.at[pl.ds(s0, SPT)])


def kernel(cost, segment_ids, num_cells, default_cost):
    bounds = jnp.arange(NW + 1, dtype=jnp.int32) * SPT
    starts = jnp.searchsorted(segment_ids, bounds, side="left").astype(jnp.int32)
    starts = jnp.pad(starts, (0, 64 - (NW + 1)))
    resid = (jnp.asarray(num_cells) - M).astype(jnp.float32)
    params = jnp.concatenate([
        jnp.full((L,), default_cost, jnp.float32),
        jnp.full((L,), resid, jnp.float32),
    ])

    mesh = plsc.VectorSubcoreMesh(
        core_axis_name="c", subcore_axis_name="s",
        num_cores=NC, num_subcores=NS)
    map_pad, mask_pad = pl.kernel(
        _sc_body,
        out_type=[
            jax.ShapeDtypeStruct((NW * SPT, G), jnp.float32),
            jax.ShapeDtypeStruct((NW * SPT,), jnp.float32),
        ],
        mesh=mesh,
        scratch_types=[
            pltpu.VMEM((SPT, G), jnp.float32),
            pltpu.VMEM((SPT + L,), jnp.float32),
            pltpu.VMEM((SPT + L,), jnp.float32),
            pltpu.VMEM((CHUNK + 2 * L,), jnp.int32),
            pltpu.VMEM((CHUNK + 2 * L,), jnp.int32),
            pltpu.VMEM((CHUNK, G), jnp.float32),
            pltpu.VMEM((CHUNK, G), jnp.float32),
            pltpu.VMEM((64,), jnp.int32),
            pltpu.VMEM((2 * L,), jnp.float32),
            pltpu.SemaphoreType.DMA,
            pltpu.SemaphoreType.DMA,
        ],
    )(cost, segment_ids, starts, params)
    return map_pad[:M], mask_pad[:M]


# speedup vs baseline: 1.0137x; 1.0087x over previous
import jax
import jax.numpy as jnp
from jax import lax
from jax.experimental import pallas as pl
from jax.experimental.pallas import tpu as pltpu
from jax.experimental.pallas import tpu_sc as plsc

N = 320000
G = 128
M = 10000

NC = 2
NS = 16
NW = NC * NS
SPT = 320
CHUNK = 256
L = 16
GV = G // L
BIG = 3.0e38


def _sc_body(cost_hbm, ids_hbm, starts_hbm, params_hbm, map_out, mask_out,
             acc, cntf, maskbuf, ids_a, ids_b, cost_a, cost_b, starts_v,
             params_v, sem_a, sem_b):
    wid = lax.axis_index("s") * NC + lax.axis_index("c")
    s0 = wid * SPT

    pltpu.sync_copy(starts_hbm, starts_v)
    pltpu.sync_copy(params_hbm, params_v)
    default_v = params_v[0:L]
    resid_v = params_v[L:2 * L]

    p0 = starts_v[pl.ds(wid, L)][0]
    p1 = starts_v[pl.ds(wid + 1, L)][0]

    big_v = jnp.full((L,), BIG, jnp.float32)
    zero_v = jnp.zeros((L,), jnp.float32)
    lane_v = lax.broadcasted_iota(jnp.int32, (L,), 0)
    one0_v = (1 - jnp.minimum(lane_v, 1)).astype(jnp.float32)

    def init_body(s, _):
        for r in range(GV):
            acc[s, pl.ds(r * L, L)] = big_v
        return 0
    lax.fori_loop(0, SPT, init_body, 0, unroll=4)

    def cnt_init(i, _):
        cntf[pl.ds(i * L, L)] = zero_v
        return 0
    lax.fori_loop(0, (SPT + L) // L, cnt_init, 0, unroll=4)

    j0 = p0 // CHUNK
    j1 = (p1 + CHUNK - 1) // CHUNK

    def run_chunk(j, idsb, costb, carry):
        off = j * CHUNK
        idsb[pl.ds(0, L)] = lax.broadcast_in_dim(carry[1], (L,), ())
        lo = jnp.maximum(p0 - off, 0)
        hi = jnp.minimum(p1, off + CHUNK) - off

        def point(li, cnt, crow):
            pv = idsb[pl.ds(L - 1 + li, L)]
            ls = pv[1] - s0
            kif = lax.convert_element_type(
                jnp.minimum(jnp.abs(pv[1] - pv[0]), 1), jnp.float32)
            kbv = lax.broadcast_in_dim(kif * BIG, (L,), ())
            new = [jnp.minimum(crow[r] + kbv,
                               costb[li, pl.ds(r * L, L)])
                   for r in range(GV)]
            for r in range(GV):
                acc[ls, pl.ds(r * L, L)] = new[r]
            ncnt = cnt * (1.0 - kif) + 1.0
            cntf[pl.ds(ls, L)] = lax.broadcast_in_dim(ncnt, (L,), ()) * one0_v
            return ncnt, new

        def pt_body(li, pc):
            ncnt, new = point(li, pc[0], pc[1:])
            return (ncnt, *new)

        def blk_body(g, pc):
            base = g * L
            idv = idsb[pl.ds(L + base, L)]
            pidv = idsb[pl.ds(L - 1 + base, L)]
            kfg = lax.convert_element_type(
                jnp.minimum(jnp.abs(idv - pidv), 1), jnp.float32)
            kbg = kfg * BIG
            cnt = pc[0]
            crow = pc[1:]
            for i in range(L):
                ls = idv[i] - s0
                kbv = lax.broadcast_in_dim(kbg[i], (L,), ())
                crow = [jnp.minimum(crow[r] + kbv,
                                    costb[base + i, pl.ds(r * L, L)])
                        for r in range(GV)]
                for r in range(GV):
                    acc[ls, pl.ds(r * L, L)] = crow[r]
                cnt = cnt * (1.0 - kfg[i]) + 1.0
                cntf[pl.ds(ls, L)] = (lax.broadcast_in_dim(cnt, (L,), ())
                                      * one0_v)
            return (cnt, *crow)

        a = jnp.minimum(hi, ((lo + L - 1) // L) * L)
        b = jnp.maximum(a, (hi // L) * L)
        pc = (carry[0],) + carry[2:]
        pc = lax.fori_loop(lo, a, pt_body, pc)
        pc = lax.fori_loop(a // L, b // L, blk_body, pc)
        pc = lax.fori_loop(b, hi, pt_body, pc)
        nprev = idsb[pl.ds(CHUNK, L)][L - 1]
        return (pc[0], nprev) + pc[1:]

    def clamp_off(j):
        return jnp.clip(j, 0, jnp.maximum(j1 - 1, 0)) * CHUNK

    def issue(j, idsb, costb, sem):
        off = clamp_off(j)
        pltpu.async_copy(ids_hbm.at[pl.ds(off, CHUNK)],
                         idsb.at[pl.ds(L, CHUNK)], sem)
        pltpu.async_copy(cost_hbm.at[pl.ds(off, CHUNK)], costb, sem)

    def drain(j, idsb, costb, sem):
        off = clamp_off(j)
        pltpu.make_async_copy(ids_hbm.at[pl.ds(off, CHUNK)],
                              idsb.at[pl.ds(L, CHUNK)], sem).wait()
        pltpu.make_async_copy(cost_hbm.at[pl.ds(off, CHUNK)], costb,
                              sem).wait()

    issue(j0, ids_a, cost_a, sem_a)
    issue(j0 + 1, ids_b, cost_b, sem_b)

    def dep0(pc):
        return lax.convert_element_type(
            jnp.minimum(jnp.abs(pc[2][0]), 0.0), jnp.int32)

    def pair_body(t, pc):
        j = j0 + 2 * t
        drain(j, ids_a, cost_a, sem_a)
        pc = run_chunk(j, ids_a, cost_a, pc)
        issue(j + 2 + dep0(pc), ids_a, cost_a, sem_a)
        drain(j + 1, ids_b, cost_b, sem_b)
        pc = run_chunk(j + 1, ids_b, cost_b, pc)
        issue(j + 3 + dep0(pc), ids_b, cost_b, sem_b)
        return pc

    carry0 = (jnp.float32(0.0), jnp.int32(-1)) + (big_v,) * GV
    nt = (j1 - j0 + 1) // 2
    pc = lax.fori_loop(0, nt, pair_body, carry0)
    jend = j0 + 2 * nt
    drain(jend, ids_a, cost_a, sem_a)
    drain(jend + 1, ids_b, cost_b, sem_b)

    def mask_body(i, _):
        cv = cntf[pl.ds(i * L, L)]
        maskbuf[pl.ds(i * L, L)] = cv - 1.0 + resid_v
        return 0
    lax.fori_loop(0, SPT // L, mask_body, 0, unroll=4)

    def sel_body(s, _):
        mv = maskbuf[pl.ds(s, L)]
        kf = jnp.clip(lax.broadcast_in_dim(mv[0], (L,), ()) + 1.0, 0.0, 1.0)
        for r in range(GV):
            a = acc[s, pl.ds(r * L, L)]
            acc[s, pl.ds(r * L, L)] = kf * a + (1.0 - kf) * default_v
        return 0
    lax.fori_loop(0, SPT, sel_body, 0, unroll=2)

    pltpu.sync_copy(acc, map_out.at[pl.ds(s0, SPT)])
    pltpu.sync_copy(maskbuf.at[pl.ds(0, SPT)], mask_out.at[pl.ds(s0, SPT)])


def kernel(cost, segment_ids, num_cells, default_cost):
    bounds = jnp.arange(NW + 1, dtype=jnp.int32) * SPT
    starts = jnp.searchsorted(segment_ids, bounds, side="left").astype(jnp.int32)
    starts = jnp.pad(starts, (0, 64 - (NW + 1)))
    resid = (jnp.asarray(num_cells) - M).astype(jnp.float32)
    params = jnp.concatenate([
        jnp.full((L,), default_cost, jnp.float32),
        jnp.full((L,), resid, jnp.float32),
    ])

    mesh = plsc.VectorSubcoreMesh(
        core_axis_name="c", subcore_axis_name="s",
        num_cores=NC, num_subcores=NS)
    map_pad, mask_pad = pl.kernel(
        _sc_body,
        out_type=[
            jax.ShapeDtypeStruct((NW * SPT, G), jnp.float32),
            jax.ShapeDtypeStruct((NW * SPT,), jnp.float32),
        ],
        mesh=mesh,
        scratch_types=[
            pltpu.VMEM((SPT, G), jnp.float32),
            pltpu.VMEM((SPT + L,), jnp.float32),
            pltpu.VMEM((SPT + L,), jnp.float32),
            pltpu.VMEM((CHUNK + 2 * L,), jnp.int32),
            pltpu.VMEM((CHUNK + 2 * L,), jnp.int32),
            pltpu.VMEM((CHUNK, G), jnp.float32),
            pltpu.VMEM((CHUNK, G), jnp.float32),
            pltpu.VMEM((64,), jnp.int32),
            pltpu.VMEM((2 * L,), jnp.float32),
            pltpu.SemaphoreType.DMA,
            pltpu.SemaphoreType.DMA,
        ],
    )(cost, segment_ids, starts, params)
    return map_pad[:M], mask_pad[:M]
